# baseline (speedup 1.0000x reference)
import functools

import jax
import jax.numpy as jnp
from jax import lax
from jax.experimental import pallas as pl
from jax.experimental.pallas import tpu as pltpu
from jax.experimental.pallas import tpu_sc as plsc

SIZE = 1000000
DIM = 32
B = 4096
L = 52
NLANES = 16

NC = 2
NS = 16
NW = NC * NS
BPW = B // NW
C = 8
CL = C * L
NCHUNK = BPW // C
NPAIR = NCHUNK // 2
JBASES = (0, 16, 32, 35)
GSLICES = ((0, 128), (128, 128), (256, 128), (384, 32))

_mesh = plsc.VectorSubcoreMesh(
    core_axis_name="c", subcore_axis_name="s", num_cores=NC, num_subcores=NS
)


@functools.partial(
    pl.kernel,
    out_type=jax.ShapeDtypeStruct((B, L - 1), jnp.float32),
    mesh=_mesh,
    scratch_types=[
        pltpu.VMEM((BPW * L,), jnp.int32),
        pltpu.VMEM((CL,), jnp.int32),
        pltpu.VMEM((CL,), jnp.int32),
        pltpu.VMEM((CL, 128), jnp.float32),
        pltpu.VMEM((CL, 128), jnp.float32),
        pltpu.VMEM((C, L - 1), jnp.float32),
        pltpu.SemaphoreType.DMA,
        pltpu.SemaphoreType.DMA,
    ],
    compiler_params=pltpu.CompilerParams(
        needs_layout_passes=False, use_tc_tiling_on_sc=True
    ),
)
def _sc_embed_dist(
    inputs_hbm, table_hbm, out_hbm,
    idx_all, q_a, q_b, rows_a, rows_b, out_v, sem_a, sem_b,
):
    wid = lax.axis_index("s") * NC + lax.axis_index("c")
    base = wid * BPW
    pltpu.sync_copy(inputs_hbm.at[pl.ds(base * L, BPW * L)], idx_all)

    iota = lax.iota(jnp.int32, NLANES)

    def qfill(k, qref):
        for t in range(CL // NLANES):
            v = idx_all[pl.ds(k * CL + t * NLANES, NLANES)]
            qref[pl.ds(t * NLANES, NLANES)] = lax.shift_right_logical(v, 2)

    def fire(qref, rows_ref, sem):
        for o, n in GSLICES:
            pltpu.make_async_copy(
                table_hbm.at[qref.at[pl.ds(o, n)]],
                rows_ref.at[pl.ds(o, n)],
                sem,
            ).start()

    def drain(qref, rows_ref, sem):
        for o, n in GSLICES:
            pltpu.make_async_copy(
                table_hbm.at[qref.at[pl.ds(o, n)]],
                rows_ref.at[pl.ds(o, n)],
                sem,
            ).wait()

    def compute(k, rows_ref, out_hbm_row):
        def row_body(r, carry):
            roff = r * L
            av = idx_all[pl.ds(k * CL + roff, NLANES)]
            cb0 = (av[0] & 3) * 32
            s0 = rows_ref[roff, pl.ds(cb0, NLANES)]
            s1 = rows_ref[roff, pl.ds(cb0 + NLANES, NLANES)]
            ridx, colb, accs = [], [], []
            for jb in JBASES:
                pos = roff + 1 + jb + iota
                idx_j = plsc.load_gather(idx_all, [k * CL + pos])
                ridx.append(pos)
                colb.append(lax.shift_left(idx_j & 3, 5))
                accs.append(jnp.zeros((NLANES,), jnp.float32))
            for d in range(DIM):
                half = s0 if d < NLANES else s1
                sb = lax.broadcast(half[d % NLANES], (NLANES,))
                for g in range(len(JBASES)):
                    v = plsc.load_gather(rows_ref, [ridx[g], colb[g] + d])
                    diff = v - sb
                    accs[g] = accs[g] + diff * diff
            for g, jb in enumerate(JBASES):
                out_v[r, pl.ds(jb, NLANES)] = -accs[g]
            return carry

        lax.fori_loop(0, C, row_body, 0)
        pltpu.sync_copy(out_v, out_hbm.at[pl.ds(out_hbm_row, C)])

    qfill(0, q_a)
    fire(q_a, rows_a, sem_a)

    def pair_body(i, carry):
        k0 = 2 * i
        k1 = 2 * i + 1
        qfill(k1, q_b)
        fire(q_b, rows_b, sem_b)
        drain(q_a, rows_a, sem_a)
        compute(k0, rows_a, base + k0 * C)
        qfill(k1 + 1, q_a)
        fire(q_a, rows_a, sem_a)
        drain(q_b, rows_b, sem_b)
        compute(k1, rows_b, base + k1 * C)
        return carry

    lax.fori_loop(0, NPAIR - 1, pair_body, 0)

    qfill(NCHUNK - 1, q_b)
    fire(q_b, rows_b, sem_b)
    drain(q_a, rows_a, sem_a)
    compute(NCHUNK - 2, rows_a, base + (NCHUNK - 2) * C)
    drain(q_b, rows_b, sem_b)
    compute(NCHUNK - 1, rows_b, base + (NCHUNK - 1) * C)


def kernel(inputs, table):
    table_packed = jnp.reshape(table, (SIZE // 4, 128))
    inputs_flat = jnp.reshape(inputs, (B * L,))
    return _sc_embed_dist(inputs_flat, table_packed)
